# baseline (speedup 1.0000x reference)
import functools

import jax
import jax.numpy as jnp
from jax import lax
from jax.experimental import pallas as pl
from jax.experimental.pallas import tpu as pltpu, tpu_sc as plsc

_info = plsc.get_sparse_core_info()
_NC, _NS = _info.num_cores, _info.num_subcores
_NW = _NC * _NS

_NB = 4096
_SEQ = 50
_B = _NB * _SEQ
_D = 128
_BPW = _B // _NW
_C = 320
_S = _BPW // _C


@functools.partial(
    pl.kernel,
    mesh=plsc.VectorSubcoreMesh(core_axis_name="c", subcore_axis_name="s"),
    out_type=jax.ShapeDtypeStruct((_B, _D), jnp.float32),
    scratch_types=[
        pltpu.VMEM((_BPW,), jnp.int32),
        pltpu.VMEM((_C, _D), jnp.float32),
        pltpu.VMEM((_C, _D), jnp.float32),
        pltpu.VMEM((_C, _D), jnp.float32),
        pltpu.SemaphoreType.DMA,
        pltpu.SemaphoreType.DMA,
        pltpu.SemaphoreType.DMA,
        pltpu.SemaphoreType.DMA,
        pltpu.SemaphoreType.DMA,
        pltpu.SemaphoreType.DMA,
    ],
)
def _lookup(table_hbm, idx_hbm, out_hbm, idx_v,
            rows0, rows1, rows2, g0, g1, g2, s0, s1, s2):
    wid = lax.axis_index("s") * _NC + lax.axis_index("c")
    base = wid * _BPW
    pltpu.sync_copy(idx_hbm.at[pl.ds(base, _BPW)], idx_v)

    rows = [rows0, rows1, rows2]
    gsem = [g0, g1, g2]
    ssem = [s0, s1, s2]

    def gather(t, b):
        return pltpu.async_copy(
            table_hbm.at[idx_v.at[pl.ds(t * _C, _C)]], rows[b], gsem[b]
        )

    def scatter(t, b):
        return pltpu.async_copy(
            rows[b], out_hbm.at[pl.ds(base + t * _C, _C)], ssem[b]
        )

    gcp = [gather(0, 0), gather(1, 1), None]
    scp = [None, None, None]
    for t in range(_S):
        b = t % 3
        gcp[b].wait()
        scp[b] = scatter(t, b)
        if t + 2 < _S:
            bb = (t + 2) % 3
            if t >= 1:
                scp[bb].wait()
            gcp[bb] = gather(t + 2, bb)
    for t in range(_S - 3, _S):
        scp[t % 3].wait()


def kernel(inputs, embedding_weights):
    idx_t = inputs.T.reshape(-1).astype(jnp.int32)
    out_t = _lookup(embedding_weights, idx_t)
    return out_t.reshape(_SEQ, _NB, _D).transpose(1, 0, 2)


# speedup vs baseline: 1.0205x; 1.0205x over previous
import functools

import jax
import jax.numpy as jnp
from jax import lax
from jax.experimental import pallas as pl
from jax.experimental.pallas import tpu as pltpu, tpu_sc as plsc

_info = plsc.get_sparse_core_info()
_NC, _NS = _info.num_cores, _info.num_subcores
_NW = _NC * _NS

_NB = 4096
_SEQ = 50
_B = _NB * _SEQ
_D = 128
_BPW = _B // _NW
_C = 400
_S = _BPW // _C


@functools.partial(
    pl.kernel,
    mesh=plsc.VectorSubcoreMesh(core_axis_name="c", subcore_axis_name="s"),
    out_type=jax.ShapeDtypeStruct((_B, _D), jnp.float32),
    compiler_params=pltpu.CompilerParams(skip_device_barrier=True),
    scratch_types=[
        pltpu.VMEM((_BPW,), jnp.int32),
        pltpu.VMEM((_C, _D), jnp.float32),
        pltpu.VMEM((_C, _D), jnp.float32),
        pltpu.SemaphoreType.DMA,
        pltpu.SemaphoreType.DMA,
    ],
)
def _lookup(table_hbm, idx_hbm, out_hbm, idx_v, rows0, rows1, sem0, sem1):
    wid = lax.axis_index("s") * _NC + lax.axis_index("c")
    base = wid * _BPW
    pltpu.sync_copy(idx_hbm.at[pl.ds(base, _BPW)], idx_v)

    rows = [rows0, rows1]
    sems = [sem0, sem1]
    cps = [
        pltpu.async_copy(table_hbm.at[idx_v.at[pl.ds(0, _C)]], rows[0], sems[0]),
        pltpu.async_copy(table_hbm.at[idx_v.at[pl.ds(_C, _C)]], rows[1], sems[1]),
    ]
    for t in range(_S):
        b = t % 2
        cps[b].wait()
        pltpu.sync_copy(rows[b], out_hbm.at[pl.ds(base + t * _C, _C)])
        if t + 2 < _S:
            cps[b] = pltpu.async_copy(
                table_hbm.at[idx_v.at[pl.ds((t + 2) * _C, _C)]], rows[b], sems[b]
            )


def kernel(inputs, embedding_weights):
    idx_t = inputs.T.reshape(-1).astype(jnp.int32)
    out_t = _lookup(embedding_weights, idx_t)
    return out_t.reshape(_SEQ, _NB, _D).transpose(1, 0, 2)


# speedup vs baseline: 1.0281x; 1.0074x over previous
import functools

import jax
import jax.numpy as jnp
from jax import lax
from jax.experimental import pallas as pl
from jax.experimental.pallas import tpu as pltpu, tpu_sc as plsc

_info = plsc.get_sparse_core_info()
_NC, _NS = _info.num_cores, _info.num_subcores
_NW = _NC * _NS

_NB = 4096
_SEQ = 50
_B = _NB * _SEQ
_D = 128
_BPW = _B // _NW
_C = 400
_S = _BPW // _C


@functools.partial(
    pl.kernel,
    mesh=plsc.VectorSubcoreMesh(core_axis_name="c", subcore_axis_name="s"),
    out_type=jax.ShapeDtypeStruct((_B, _D), jnp.float32),
    scratch_types=[
        pltpu.VMEM((_BPW,), jnp.int32),
        pltpu.VMEM((_C, _D), jnp.float32),
        pltpu.VMEM((_C, _D), jnp.float32),
        pltpu.SemaphoreType.DMA,
        pltpu.SemaphoreType.DMA,
    ],
)
def _lookup(table_hbm, idx_hbm, out_hbm, idx_v, rows0, rows1, sem0, sem1):
    wid = lax.axis_index("s") * _NC + lax.axis_index("c")
    base = wid * _BPW
    pltpu.sync_copy(idx_hbm.at[pl.ds(base, _BPW)], idx_v)

    rows = [rows0, rows1]
    sems = [sem0, sem1]

    def gather(t, b):
        pltpu.async_copy(table_hbm.at[idx_v.at[pl.ds(t * _C, _C)]], rows[b], sems[b])

    def step(t, b):
        pltpu.make_async_copy(out_hbm.at[pl.ds(base, _C)], rows[b], sems[b]).wait()
        pltpu.sync_copy(rows[b], out_hbm.at[pl.ds(base + t * _C, _C)])

    gather(0, 0)
    gather(1, 1)

    def body(i, carry):
        t0 = 2 * i
        for j in range(2):
            step(t0 + j, j)
            gather(t0 + j + 2, j)
        return carry

    lax.fori_loop(0, (_S - 2) // 2, body, 0)
    step(_S - 2, 0)
    step(_S - 1, 1)


def kernel(inputs, embedding_weights):
    idx_t = inputs.T.reshape(-1).astype(jnp.int32)
    out_t = _lookup(embedding_weights, idx_t)
    return out_t.reshape(_SEQ, _NB, _D).transpose(1, 0, 2)


# speedup vs baseline: 1.0286x; 1.0005x over previous
import functools

import jax
import jax.numpy as jnp
from jax import lax
from jax.experimental import pallas as pl
from jax.experimental.pallas import tpu as pltpu, tpu_sc as plsc

_info = plsc.get_sparse_core_info()
_NC, _NS = _info.num_cores, _info.num_subcores
_NW = _NC * _NS

_NB = 4096
_SEQ = 50
_B = _NB * _SEQ
_D = 128
_BPW = _B // _NW
_C = 400
_S = _BPW // _C


@functools.partial(
    pl.kernel,
    mesh=plsc.VectorSubcoreMesh(core_axis_name="c", subcore_axis_name="s"),
    out_type=jax.ShapeDtypeStruct((_B, _D), jnp.float32),
    scratch_types=[
        pltpu.VMEM((_BPW,), jnp.int32),
        pltpu.VMEM((_C, _D), jnp.float32),
        pltpu.VMEM((_C, _D), jnp.float32),
        pltpu.SemaphoreType.DMA,
        pltpu.SemaphoreType.DMA,
    ],
)
def _lookup(table_hbm, idx_hbm, out_hbm, idx_v, rows0, rows1, sem0, sem1):
    wid = lax.axis_index("s") * _NC + lax.axis_index("c")
    base = wid * _BPW
    pltpu.sync_copy(idx_hbm.at[pl.ds(base, _BPW)], idx_v)

    rows = [rows0, rows1]
    sems = [sem0, sem1]

    _H = _C // 2

    def gather(t, b):
        pltpu.async_copy(
            table_hbm.at[idx_v.at[pl.ds(t * _C, _H)]],
            rows[b].at[pl.ds(0, _H)], sems[b]
        )
        pltpu.async_copy(
            table_hbm.at[idx_v.at[pl.ds(t * _C + _H, _H)]],
            rows[b].at[pl.ds(_H, _H)], sems[b]
        )

    def step(t, b):
        pltpu.make_async_copy(out_hbm.at[pl.ds(base, _C)], rows[b], sems[b]).wait()
        pltpu.sync_copy(rows[b], out_hbm.at[pl.ds(base + t * _C, _C)])

    gather(0, 0)
    gather(1, 1)

    def body(i, carry):
        t0 = 2 * i
        for j in range(2):
            step(t0 + j, j)
            gather(t0 + j + 2, j)
        return carry

    lax.fori_loop(0, (_S - 2) // 2, body, 0)
    step(_S - 2, 0)
    step(_S - 1, 1)


def kernel(inputs, embedding_weights):
    idx_t = inputs.T.reshape(-1).astype(jnp.int32)
    out_t = _lookup(embedding_weights, idx_t)
    return out_t.reshape(_SEQ, _NB, _D).transpose(1, 0, 2)
